# Initial kernel scaffold; baseline (speedup 1.0000x reference)
import jax
import jax.numpy as jnp
from jax.experimental import pallas as pl


def kernel(inputs, states):
    raise NotImplementedError("write your pallas kernel here")


# speedup vs baseline: 4.2557x; 4.2557x over previous
import functools

import jax
import jax.numpy as jnp
import numpy as np
from jax.experimental import pallas as pl

N = 14
L = 128
NN = N * N
NNN = N * N * N


def _dft_constants():
    n = np.arange(L)[:, None].astype(np.float64)
    k = np.arange(L)[None, :].astype(np.float64)
    ang = 2.0 * np.pi * n * k / L
    live_f = (k <= 64)
    fc = np.where(live_f, np.cos(ang), 0.0)
    fs = np.where((k >= 1) & (k <= 63), -np.sin(ang), 0.0)
    kk = np.arange(L)[:, None].astype(np.float64)
    ll = np.arange(L)[None, :].astype(np.float64)
    ang2 = 2.0 * np.pi * kk * ll / L
    w = np.where(kk == 0, 1.0, np.where(kk == 64, 1.0, np.where(kk <= 63, 2.0, 0.0)))
    gc = w * np.cos(ang2) / L
    gs = np.where((kk >= 1) & (kk <= 63), -2.0 * np.sin(ang2) / L, 0.0)
    pt = np.zeros((NN, NN))
    for a in range(N):
        for b in range(N):
            pt[a * N + b, b * N + a] = 1.0
    pr = np.zeros((L, L))
    for l in range(L):
        pr[(L - l) % L, l] = 1.0
    return (jnp.asarray(fc, jnp.float32), jnp.asarray(fs, jnp.float32),
            jnp.asarray(gc, jnp.float32), jnp.asarray(gs, jnp.float32),
            jnp.asarray(pt, jnp.float32), jnp.asarray(pr, jnp.float32))


def _body(inp_ref, st_ref, fc_ref, fs_ref, gc_ref, gs_ref, pt_ref, pr_ref,
          out_o_ref, out_s_ref):
    f32 = jnp.float32
    x0 = inp_ref[0]
    st = st_ref[0]
    rows = jax.lax.broadcasted_iota(jnp.int32, (NN, 1), 0)
    lanes = jax.lax.broadcasted_iota(jnp.int32, (NN, L), 1)
    is_diag = (rows % (N + 1)) == 0
    dirac = jnp.where(lanes == 0, 1.0, 0.0).astype(f32)
    z = jnp.where(is_diag, dirac, x0 * st)
    x = z / jnp.sum(z, axis=-1, keepdims=True)

    dot = functools.partial(jax.lax.dot, preferred_element_type=f32)
    sr = dot(x, fc_ref[...])
    si = dot(x, fs_ref[...])

    t1r = jnp.broadcast_to(sr.reshape(N, 1, N, L), (N, N, N, L)).reshape(NNN, L)
    t1i = jnp.broadcast_to(si.reshape(N, 1, N, L), (N, N, N, L)).reshape(NNN, L)
    t2r = jnp.broadcast_to(sr.reshape(N, N, 1, L), (N, N, N, L)).reshape(NNN, L)
    t2i = jnp.broadcast_to(si.reshape(N, N, 1, L), (N, N, N, L)).reshape(NNN, L)
    prr = t1r * t2r + t1i * t2i
    pii = t1i * t2r - t1r * t2i
    comb = dot(prr, gc_ref[...]) + dot(pii, gs_ref[...])

    r = jax.lax.broadcasted_iota(jnp.int32, (NNN, 1), 0)
    bb = r // NN
    aa = (r // N) % N
    cc = r % N
    comb = jnp.where((bb == aa) | (bb == cc), 1.0, comb)

    comb3 = comb.reshape(N, NN, L)
    m = comb3[0]
    for b in range(1, N):
        m = m * comb3[b]
    out_s_ref[0] = m / jnp.sum(m, axis=-1, keepdims=True)

    xt = dot(pt_ref[...], x)
    rev = dot(xt, pr_ref[...])
    u = m * x * rev
    u = jnp.where(is_diag, dirac, u)
    out_o_ref[0] = u / jnp.sum(u, axis=-1, keepdims=True)


def kernel(inputs, states):
    B = inputs.shape[0]
    fc, fs, gc, gs, pt, pr = _dft_constants()
    full = lambda s: pl.BlockSpec(s, lambda i: (0, 0))
    blk = pl.BlockSpec((1, NN, L), lambda i: (i, 0, 0))
    out_o, out_s = pl.pallas_call(
        _body,
        grid=(B,),
        in_specs=[blk, blk, full((L, L)), full((L, L)), full((L, L)),
                  full((L, L)), full((NN, NN)), full((L, L))],
        out_specs=[blk, blk],
        out_shape=[jax.ShapeDtypeStruct((B, NN, L), jnp.float32),
                   jax.ShapeDtypeStruct((B, NN, L), jnp.float32)],
    )(inputs, states, fc, fs, gc, gs, pt, pr)
    return out_o.reshape(B, N, N, L), out_s


# speedup vs baseline: 8.2574x; 1.9403x over previous
import functools

import jax
import jax.numpy as jnp
import numpy as np
from jax.experimental import pallas as pl

N = 14
L = 128
NN = N * N
NNN = N * N * N
NP = 16
NPP = NP * NP
NR = N * NP * NP


def _dft_constants():
    n = np.arange(L)[:, None].astype(np.float64)
    k = np.arange(L)[None, :].astype(np.float64)
    ang = 2.0 * np.pi * n * k / L
    live_f = (k <= 64)
    fc = np.where(live_f, np.cos(ang), 0.0)
    fs = np.where((k >= 1) & (k <= 63), -np.sin(ang), 0.0)
    kk = np.arange(L)[:, None].astype(np.float64)
    ll = np.arange(L)[None, :].astype(np.float64)
    ang2 = 2.0 * np.pi * kk * ll / L
    w = np.where(kk == 0, 1.0, np.where(kk == 64, 1.0, np.where(kk <= 63, 2.0, 0.0)))
    gc = w * np.cos(ang2) / L
    gs = np.where((kk >= 1) & (kk <= 63), -2.0 * np.sin(ang2) / L, 0.0)
    pt = np.zeros((NN, NN))
    for a in range(N):
        for b in range(N):
            pt[a * N + b, b * N + a] = 1.0
    pr = np.zeros((L, L))
    for l in range(L):
        pr[(L - l) % L, l] = 1.0
    ppad = np.zeros((N * NP, NN))
    for b in range(N):
        for s in range(N):
            ppad[b * NP + s, b * N + s] = 1.0
    pext = np.zeros((NN, NP * NP))
    for a in range(N):
        for c in range(N):
            pext[a * N + c, a * NP + c] = 1.0
    return (jnp.asarray(fc, jnp.float32), jnp.asarray(fs, jnp.float32),
            jnp.asarray(gc, jnp.float32), jnp.asarray(gs, jnp.float32),
            jnp.asarray(pt, jnp.float32), jnp.asarray(pr, jnp.float32),
            jnp.asarray(ppad, jnp.float32), jnp.asarray(pext, jnp.float32))


def _body(inp_ref, st_ref, fc_ref, fs_ref, gc_ref, gs_ref, pt_ref, pr_ref,
          ppad_ref, pext_ref, out_o_ref, out_s_ref):
    f32 = jnp.float32
    x0 = inp_ref[0]
    st = st_ref[0]
    rows = jax.lax.broadcasted_iota(jnp.int32, (NN, 1), 0)
    lanes = jax.lax.broadcasted_iota(jnp.int32, (NN, L), 1)
    is_diag = (rows % (N + 1)) == 0
    dirac = jnp.where(lanes == 0, 1.0, 0.0).astype(f32)
    z = jnp.where(is_diag, dirac, x0 * st)
    x = z / jnp.sum(z, axis=-1, keepdims=True)

    dot = functools.partial(jax.lax.dot, preferred_element_type=f32)
    xp = dot(ppad_ref[...], x)
    sr = dot(xp, fc_ref[...])
    si = dot(xp, fs_ref[...])

    t1r = jnp.broadcast_to(sr.reshape(N, 1, NP, L), (N, NP, NP, L)).reshape(NR, L)
    t1i = jnp.broadcast_to(si.reshape(N, 1, NP, L), (N, NP, NP, L)).reshape(NR, L)
    t2r = jnp.broadcast_to(sr.reshape(N, NP, 1, L), (N, NP, NP, L)).reshape(NR, L)
    t2i = jnp.broadcast_to(si.reshape(N, NP, 1, L), (N, NP, NP, L)).reshape(NR, L)
    prr = t1r * t2r + t1i * t2i
    pii = t1i * t2r - t1r * t2i
    comb = dot(prr, gc_ref[...]) + dot(pii, gs_ref[...])

    r = jax.lax.broadcasted_iota(jnp.int32, (NR, 1), 0)
    bb = r // NPP
    aa = (r // NP) % NP
    cc = r % NP
    comb = jnp.where((bb == aa) | (bb == cc) | (aa >= N) | (cc >= N), 1.0, comb)

    comb3 = comb.reshape(N, NPP, L)
    mp = comb3[0]
    for b in range(1, N):
        mp = mp * comb3[b]
    m = dot(pext_ref[...], mp)
    out_s_ref[0] = m / jnp.sum(m, axis=-1, keepdims=True)

    xt = dot(pt_ref[...], x)
    rev = dot(xt, pr_ref[...])
    u = m * x * rev
    u = jnp.where(is_diag, dirac, u)
    out_o_ref[0] = u / jnp.sum(u, axis=-1, keepdims=True)


def kernel(inputs, states):
    B = inputs.shape[0]
    fc, fs, gc, gs, pt, pr, ppad, pext = _dft_constants()
    full = lambda s: pl.BlockSpec(s, lambda i: (0, 0))
    blk = pl.BlockSpec((1, NN, L), lambda i: (i, 0, 0))
    out_o, out_s = pl.pallas_call(
        _body,
        grid=(B,),
        in_specs=[blk, blk, full((L, L)), full((L, L)), full((L, L)),
                  full((L, L)), full((NN, NN)), full((L, L)),
                  full((N * NP, NN)), full((NN, NPP))],
        out_specs=[blk, blk],
        out_shape=[jax.ShapeDtypeStruct((B, NN, L), jnp.float32),
                   jax.ShapeDtypeStruct((B, NN, L), jnp.float32)],
    )(inputs, states, fc, fs, gc, gs, pt, pr, ppad, pext)
    return out_o.reshape(B, N, N, L), out_s


# speedup vs baseline: 8.3779x; 1.0146x over previous
import functools

import jax
import jax.numpy as jnp
import numpy as np
from jax.experimental import pallas as pl

N = 14
L = 128
NN = N * N
NNN = N * N * N
NP = 16
NPP = NP * NP
NR = N * NP * NP


def _dft_constants():
    n = np.arange(L)[:, None].astype(np.float64)
    k = np.arange(L)[None, :].astype(np.float64)
    ang = 2.0 * np.pi * n * k / L
    live_f = (k <= 64)
    fc = np.where(live_f, np.cos(ang), 0.0)
    fs = np.where((k >= 1) & (k <= 63), -np.sin(ang), 0.0)
    kk = np.arange(L)[:, None].astype(np.float64)
    ll = np.arange(L)[None, :].astype(np.float64)
    ang2 = 2.0 * np.pi * kk * ll / L
    w = np.where(kk == 0, 1.0, np.where(kk == 64, 1.0, np.where(kk <= 63, 2.0, 0.0)))
    gc = w * np.cos(ang2) / L
    gs = np.where((kk >= 1) & (kk <= 63), -2.0 * np.sin(ang2) / L, 0.0)
    pt = np.zeros((NN, NN))
    for a in range(N):
        for b in range(N):
            pt[a * N + b, b * N + a] = 1.0
    pr = np.zeros((L, L))
    for l in range(L):
        pr[(L - l) % L, l] = 1.0
    ppad = np.zeros((N * NP, NN))
    for b in range(N):
        for s in range(N):
            if s != b:
                ppad[b * NP + s, b * N + s] = 1.0
    fc[:, 66] = 1.0
    gc[66, :] = -1.0
    pext = np.zeros((NN, NP * NP))
    for a in range(N):
        for c in range(N):
            pext[a * N + c, a * NP + c] = 1.0
    return (jnp.asarray(fc, jnp.float32), jnp.asarray(fs, jnp.float32),
            jnp.asarray(gc, jnp.float32), jnp.asarray(gs, jnp.float32),
            jnp.asarray(pt, jnp.float32), jnp.asarray(pr, jnp.float32),
            jnp.asarray(ppad, jnp.float32), jnp.asarray(pext, jnp.float32))


def _body(inp_ref, st_ref, fc_ref, fs_ref, gc_ref, gs_ref, pt_ref, pr_ref,
          ppad_ref, pext_ref, out_o_ref, out_s_ref):
    f32 = jnp.float32
    x0 = inp_ref[0]
    st = st_ref[0]
    rows = jax.lax.broadcasted_iota(jnp.int32, (NN, 1), 0)
    lanes = jax.lax.broadcasted_iota(jnp.int32, (NN, L), 1)
    is_diag = (rows % (N + 1)) == 0
    dirac = jnp.where(lanes == 0, 1.0, 0.0).astype(f32)
    z = jnp.where(is_diag, dirac, x0 * st)

    dot = functools.partial(jax.lax.dot, preferred_element_type=f32)
    norm = lambda v: v * (1.0 / jnp.sum(v, axis=-1, keepdims=True))
    x = norm(z)
    xp = dot(ppad_ref[...], x)
    sr = dot(xp, fc_ref[...])
    si = dot(xp, fs_ref[...])

    t1r = jnp.broadcast_to(sr.reshape(N, 1, NP, L), (N, NP, NP, L)).reshape(NR, L)
    t1i = jnp.broadcast_to(si.reshape(N, 1, NP, L), (N, NP, NP, L)).reshape(NR, L)
    t2r = jnp.broadcast_to(sr.reshape(N, NP, 1, L), (N, NP, NP, L)).reshape(NR, L)
    t2i = jnp.broadcast_to(si.reshape(N, NP, 1, L), (N, NP, NP, L)).reshape(NR, L)
    prr = t1r * t2r + t1i * t2i
    pii = t1i * t2r - t1r * t2i
    comb = dot(prr, gc_ref[...]) + dot(pii, gs_ref[...]) + 1.0

    comb3 = comb.reshape(N, NPP, L)
    mp = comb3[0]
    for b in range(1, N):
        mp = mp * comb3[b]
    m = dot(pext_ref[...], mp)
    out_s_ref[0] = norm(m)

    xt = dot(pt_ref[...], x)
    rev = dot(xt, pr_ref[...])
    u = m * x * rev
    u = jnp.where(is_diag, dirac, u)
    out_o_ref[0] = norm(u)


def kernel(inputs, states):
    B = inputs.shape[0]
    fc, fs, gc, gs, pt, pr, ppad, pext = _dft_constants()
    full = lambda s: pl.BlockSpec(s, lambda i: (0, 0))
    blk = pl.BlockSpec((1, NN, L), lambda i: (i, 0, 0))
    out_o, out_s = pl.pallas_call(
        _body,
        grid=(B,),
        in_specs=[blk, blk, full((L, L)), full((L, L)), full((L, L)),
                  full((L, L)), full((NN, NN)), full((L, L)),
                  full((N * NP, NN)), full((NN, NPP))],
        out_specs=[blk, blk],
        out_shape=[jax.ShapeDtypeStruct((B, NN, L), jnp.float32),
                   jax.ShapeDtypeStruct((B, NN, L), jnp.float32)],
    )(inputs, states, fc, fs, gc, gs, pt, pr, ppad, pext)
    return out_o.reshape(B, N, N, L), out_s


# speedup vs baseline: 8.5259x; 1.0177x over previous
import functools

import jax
import jax.numpy as jnp
import numpy as np
from jax.experimental import pallas as pl

N = 14
L = 128
NN = N * N
NNN = N * N * N
NP = 16
NPP = NP * NP
NR = N * NP * NP


def _dft_constants():
    n = np.arange(L)[:, None].astype(np.float64)
    k = np.arange(L)[None, :].astype(np.float64)
    ang = 2.0 * np.pi * n * k / L
    live_f = (k <= 64)
    fc = np.where(live_f, np.cos(ang), 0.0)
    fs = np.where((k >= 1) & (k <= 63), -np.sin(ang), 0.0)
    kk = np.arange(L)[:, None].astype(np.float64)
    ll = np.arange(L)[None, :].astype(np.float64)
    ang2 = 2.0 * np.pi * kk * ll / L
    w = np.where(kk == 0, 1.0, np.where(kk == 64, 1.0, np.where(kk <= 63, 2.0, 0.0)))
    gc = w * np.cos(ang2) / L
    gs = np.where((kk >= 1) & (kk <= 63), -2.0 * np.sin(ang2) / L, 0.0)
    pt = np.zeros((NN, NN))
    for a in range(N):
        for b in range(N):
            pt[a * N + b, b * N + a] = 1.0
    pr = np.zeros((L, L))
    for l in range(L):
        pr[(L - l) % L, l] = 1.0
    ppad = np.zeros((N * NP, NN))
    for b in range(N):
        for s in range(N):
            if s != b:
                ppad[b * NP + s, b * N + s] = 1.0
    fp = np.zeros((L, L))
    fp[:, :65] = fc[:, :65]
    fp[:, 65:] = fs[:, 1:64]
    gp = np.zeros((L, L))
    gp[:65, :] = gc[:65, :]
    gp[65:, :] = gs[1:64, :]
    mk = np.zeros((NR, L))
    for b in range(N):
        for a in range(NP):
            for c in range(NP):
                if a == b or c == b or a >= N or c >= N:
                    mk[b * NPP + a * NP + c, :] = 1.0
    pext = np.zeros((NN, NP * NP))
    for a in range(N):
        for c in range(N):
            pext[a * N + c, a * NP + c] = 1.0
    return (jnp.asarray(fp, jnp.float32), jnp.asarray(gp, jnp.float32),
            jnp.asarray(pt, jnp.float32), jnp.asarray(pr, jnp.float32),
            jnp.asarray(ppad, jnp.float32), jnp.asarray(pext, jnp.float32),
            jnp.asarray(mk, jnp.float32))


def _body(inp_ref, st_ref, fp_ref, gp_ref, pt_ref, pr_ref,
          ppad_ref, pext_ref, mk_ref, out_o_ref, out_s_ref):
    f32 = jnp.float32
    x0 = inp_ref[0]
    st = st_ref[0]
    rows = jax.lax.broadcasted_iota(jnp.int32, (NN, 1), 0)
    lanes = jax.lax.broadcasted_iota(jnp.int32, (NN, L), 1)
    is_diag = (rows % (N + 1)) == 0
    dirac = jnp.where(lanes == 0, 1.0, 0.0).astype(f32)
    z = jnp.where(is_diag, dirac, x0 * st)

    dot = functools.partial(jax.lax.dot, preferred_element_type=f32)
    norm = lambda v: v * (1.0 / jnp.sum(v, axis=-1, keepdims=True))
    x = norm(z)
    xp = dot(ppad_ref[...], x)
    sp = dot(xp, fp_ref[...])
    lidx = jax.lax.broadcasted_iota(jnp.int32, (1, L), 1)
    spr = jnp.where((lidx == 0) | (lidx == 64), 0.0, jnp.roll(sp, 64, axis=-1))

    t1 = jnp.broadcast_to(sp.reshape(N, 1, NP, L), (N, NP, NP, L)).reshape(NR, L)
    t2 = jnp.broadcast_to(sp.reshape(N, NP, 1, L), (N, NP, NP, L)).reshape(NR, L)
    u1 = jnp.broadcast_to(spr.reshape(N, 1, NP, L), (N, NP, NP, L)).reshape(NR, L)
    u2 = jnp.broadcast_to(spr.reshape(N, NP, 1, L), (N, NP, NP, L)).reshape(NR, L)
    av = t1 * t2
    ar = u1 * u2
    bv = t1 * u2
    br = u1 * t2
    pk = jnp.where(lidx <= 64, av + ar, bv - br)
    comb = dot(pk, gp_ref[...]) + mk_ref[...]

    comb3 = comb.reshape(N, NPP, L)
    mp = comb3[0]
    for b in range(1, N):
        mp = mp * comb3[b]
    m = dot(pext_ref[...], mp)
    out_s_ref[0] = norm(m)

    xt = dot(pt_ref[...], x)
    rev = dot(xt, pr_ref[...])
    u = m * x * rev
    u = jnp.where(is_diag, dirac, u)
    out_o_ref[0] = norm(u)


def kernel(inputs, states):
    B = inputs.shape[0]
    fp, gp, pt, pr, ppad, pext, mk = _dft_constants()
    full = lambda s: pl.BlockSpec(s, lambda i: (0, 0))
    blk = pl.BlockSpec((1, NN, L), lambda i: (i, 0, 0))
    out_o, out_s = pl.pallas_call(
        _body,
        grid=(B,),
        in_specs=[blk, blk, full((L, L)), full((L, L)),
                  full((NN, NN)), full((L, L)),
                  full((N * NP, NN)), full((NN, NPP)), full((NR, L))],
        out_specs=[blk, blk],
        out_shape=[jax.ShapeDtypeStruct((B, NN, L), jnp.float32),
                   jax.ShapeDtypeStruct((B, NN, L), jnp.float32)],
    )(inputs, states, fp, gp, pt, pr, ppad, pext, mk)
    return out_o.reshape(B, N, N, L), out_s


# speedup vs baseline: 9.4703x; 1.1108x over previous
import functools

import jax
import jax.numpy as jnp
import numpy as np
from jax.experimental import pallas as pl

N = 14
L = 128
NN = N * N
NNN = N * N * N
NP = 16
NPP = NP * NP
NR = N * NP * NP
PB = 2


def _dft_constants():
    n = np.arange(L)[:, None].astype(np.float64)
    k = np.arange(L)[None, :].astype(np.float64)
    ang = 2.0 * np.pi * n * k / L
    live_f = (k <= 64)
    fc = np.where(live_f, np.cos(ang), 0.0)
    fs = np.where((k >= 1) & (k <= 63), -np.sin(ang), 0.0)
    kk = np.arange(L)[:, None].astype(np.float64)
    ll = np.arange(L)[None, :].astype(np.float64)
    ang2 = 2.0 * np.pi * kk * ll / L
    w = np.where(kk == 0, 1.0, np.where(kk == 64, 1.0, np.where(kk <= 63, 2.0, 0.0)))
    gc = w * np.cos(ang2) / L
    gs = np.where((kk >= 1) & (kk <= 63), -2.0 * np.sin(ang2) / L, 0.0)
    pt = np.zeros((NN, NN))
    for a in range(N):
        for b in range(N):
            pt[a * N + b, b * N + a] = 1.0
    pr = np.zeros((L, L))
    for l in range(L):
        pr[(L - l) % L, l] = 1.0
    ppad = np.zeros((N * NP, NN))
    for b in range(N):
        for s in range(N):
            if s != b:
                ppad[b * NP + s, b * N + s] = 1.0
    fp = np.zeros((L, L))
    fp[:, :65] = fc[:, :65]
    fp[:, 65:] = fs[:, 1:64]
    gp = np.zeros((L, L))
    gp[:65, :] = gc[:65, :]
    gp[65:, :] = gs[1:64, :]
    mk = np.zeros((NR, L))
    for b in range(N):
        for a in range(NP):
            for c in range(NP):
                if a == b or c == b or a >= N or c >= N:
                    mk[b * NPP + a * NP + c, :] = 1.0
    pext = np.zeros((NN, NP * NP))
    for a in range(N):
        for c in range(N):
            pext[a * N + c, a * NP + c] = 1.0
    eye = np.eye(PB)
    pt = np.kron(eye, pt)
    ppad = np.kron(eye, ppad)
    pext = np.kron(eye, pext)
    mk = np.tile(mk, (PB, 1))
    return (jnp.asarray(fp, jnp.float32), jnp.asarray(gp, jnp.float32),
            jnp.asarray(pt, jnp.float32), jnp.asarray(pr, jnp.float32),
            jnp.asarray(ppad, jnp.float32), jnp.asarray(pext, jnp.float32),
            jnp.asarray(mk, jnp.float32))


def _body(inp_ref, st_ref, fp_ref, gp_ref, pt_ref, pr_ref,
          ppad_ref, pext_ref, mk_ref, out_o_ref, out_s_ref):
    f32 = jnp.float32
    x0 = inp_ref[...]
    st = st_ref[...]
    rows = jax.lax.broadcasted_iota(jnp.int32, (1, NN, 1), 1)
    lanes = jax.lax.broadcasted_iota(jnp.int32, (1, NN, L), 2)
    is_diag = (rows % (N + 1)) == 0
    dirac = jnp.where(lanes == 0, 1.0, 0.0).astype(f32)
    z = jnp.where(is_diag, dirac, x0 * st)

    dot = functools.partial(jax.lax.dot, preferred_element_type=f32)
    norm = lambda v: v * (1.0 / jnp.sum(v, axis=-1, keepdims=True))
    x = norm(z).reshape(PB * NN, L)
    xp = dot(ppad_ref[...], x)
    sp = dot(xp, fp_ref[...])
    lidx = jax.lax.broadcasted_iota(jnp.int32, (1, L), 1)
    spr = jnp.where((lidx == 0) | (lidx == 64), 0.0, jnp.roll(sp, 64, axis=-1))

    nb = PB * N
    t1 = jnp.broadcast_to(sp.reshape(nb, 1, NP, L), (nb, NP, NP, L)).reshape(PB * NR, L)
    t2 = jnp.broadcast_to(sp.reshape(nb, NP, 1, L), (nb, NP, NP, L)).reshape(PB * NR, L)
    u1 = jnp.broadcast_to(spr.reshape(nb, 1, NP, L), (nb, NP, NP, L)).reshape(PB * NR, L)
    u2 = jnp.broadcast_to(spr.reshape(nb, NP, 1, L), (nb, NP, NP, L)).reshape(PB * NR, L)
    av = t1 * t2
    ar = u1 * u2
    bv = t1 * u2
    br = u1 * t2
    pk = jnp.where(lidx <= 64, av + ar, bv - br)
    comb = dot(pk, gp_ref[...]) + mk_ref[...]

    comb3 = comb.reshape(nb, NPP, L)
    mps = []
    for p in range(PB):
        mp = comb3[p * N]
        for b in range(1, N):
            mp = mp * comb3[p * N + b]
        mps.append(mp)
    mp = jnp.concatenate(mps, axis=0)
    m = dot(pext_ref[...], mp)
    out_s_ref[...] = norm(m.reshape(PB, NN, L))

    xt = dot(pt_ref[...], x)
    rev = dot(xt, pr_ref[...])
    u = (m * x * rev).reshape(PB, NN, L)
    u = jnp.where(is_diag, dirac, u)
    out_o_ref[...] = norm(u)


def kernel(inputs, states):
    B = inputs.shape[0]
    fp, gp, pt, pr, ppad, pext, mk = _dft_constants()
    full = lambda s: pl.BlockSpec(s, lambda i: (0, 0))
    blk = pl.BlockSpec((PB, NN, L), lambda i: (i, 0, 0))
    out_o, out_s = pl.pallas_call(
        _body,
        grid=(B // PB,),
        in_specs=[blk, blk, full((L, L)), full((L, L)),
                  full((PB * NN, PB * NN)), full((L, L)),
                  full((PB * N * NP, PB * NN)), full((PB * NN, PB * NPP)),
                  full((PB * NR, L))],
        out_specs=[blk, blk],
        out_shape=[jax.ShapeDtypeStruct((B, NN, L), jnp.float32),
                   jax.ShapeDtypeStruct((B, NN, L), jnp.float32)],
    )(inputs, states, fp, gp, pt, pr, ppad, pext, mk)
    return out_o.reshape(B, N, N, L), out_s
